# baseline (speedup 1.0000x reference)
import functools

import jax
import jax.numpy as jnp
from jax import lax
from jax.experimental import pallas as pl
from jax.experimental.pallas import tpu as pltpu
from jax.experimental.pallas import tpu_sc as plsc

N = 10000
E = 320000
D = 128

NC = 2
NS = 16
NW = NC * NS
B = 128
CH = 16
NBLK = 80
E_PAD = NW * B * NBLK
NCHUNK = NBLK // CH
N_PAD = 10240
RPT = N_PAD // NS


def _sc_segment_sum(x, src3, dst3):
    mesh = plsc.VectorSubcoreMesh(core_axis_name="c", subcore_axis_name="s")

    @functools.partial(
        pl.kernel,
        out_type=jax.ShapeDtypeStruct((NC * N_PAD, D), jnp.float32),
        mesh=mesh,
        scratch_types=dict(
            acc=pltpu.VMEM_SHARED((N_PAD, D), jnp.float32),
            sidx=pltpu.VMEM((2, CH, B), jnp.int32),
            didx=pltpu.VMEM((2, CH, B), jnp.int32),
            rows_v=pltpu.VMEM((2, B, D), jnp.float32),
            gsem=pltpu.SemaphoreType.DMA,
            isem=pltpu.SemaphoreType.DMA,
        ),
    )
    def seg_sum(x_hbm, src_hbm, dst_hbm, out_hbm, acc, sidx, didx,
                rows_v, gsem, isem):
        c = lax.axis_index("c")
        s = lax.axis_index("s")
        wid = c * NS + s

        z = jnp.zeros((16,), jnp.float32)

        def zrow(i, carry):
            for j in range(D // 16):
                rows_v[0, i, pl.ds(j * 16, 16)] = z
            return carry
        lax.fori_loop(0, B, zrow, 0)

        def zero_body(k, carry):
            pltpu.sync_copy(rows_v.at[0], acc.at[pl.ds(s * RPT + k * B, B)])
            return carry
        lax.fori_loop(0, RPT // B, zero_body, 0)

        pltpu.sync_copy(src_hbm.at[wid, pl.ds(0, CH)], sidx.at[0])
        pltpu.sync_copy(dst_hbm.at[wid, pl.ds(0, CH)], didx.at[0])

        plsc.subcore_barrier()

        def chunk_body(ci, carry):
            cb = lax.rem(ci, 2)
            nb = lax.rem(ci + 1, 2)

            @pl.when(ci + 1 < NCHUNK)
            def _():
                pltpu.async_copy(
                    src_hbm.at[wid, pl.ds((ci + 1) * CH, CH)],
                    sidx.at[nb], isem)
                pltpu.async_copy(
                    dst_hbm.at[wid, pl.ds((ci + 1) * CH, CH)],
                    didx.at[nb], isem)

            for j in range(CH):
                b0 = j % 2
                pltpu.async_copy(x_hbm.at[sidx.at[cb, j]], rows_v.at[b0],
                                 gsem).wait()
                pltpu.sync_copy(rows_v.at[b0], acc.at[didx.at[cb, j]],
                                add=True)

            @pl.when(ci + 1 < NCHUNK)
            def _():
                pltpu.make_async_copy(
                    src_hbm.at[wid, pl.ds(0, CH)], sidx.at[nb], isem).wait()
                pltpu.make_async_copy(
                    dst_hbm.at[wid, pl.ds(0, CH)], didx.at[nb], isem).wait()
            return carry
        lax.fori_loop(0, NCHUNK, chunk_body, 0)

        plsc.subcore_barrier()

        pltpu.sync_copy(acc.at[pl.ds(s * RPT, RPT)],
                        out_hbm.at[pl.ds(c * N_PAD + s * RPT, RPT)])

    return seg_sum(x, src3, dst3)


def _tc_linear(p0, p1, W):
    BN = 1000

    def body(p0_ref, p1_ref, w_ref, o_ref):
        agg = p0_ref[...] + p1_ref[...]
        o_ref[...] = lax.dot_general(
            agg, w_ref[...], (((1,), (1,)), ((), ())),
            preferred_element_type=jnp.float32)

    return pl.pallas_call(
        body,
        grid=(N // BN,),
        in_specs=[
            pl.BlockSpec((BN, D), lambda i: (i, 0)),
            pl.BlockSpec((BN, D), lambda i: (i, 0)),
            pl.BlockSpec((D, D), lambda i: (0, 0)),
        ],
        out_specs=pl.BlockSpec((BN, D), lambda i: (i, 0)),
        out_shape=jax.ShapeDtypeStruct((N, D), jnp.float32),
    )(p0, p1, W)


def kernel(x, edge_index, W):
    dst = edge_index[0]
    src = edge_index[1]
    pad = E_PAD - E
    src_p = jnp.concatenate([src, jnp.zeros((pad,), jnp.int32)])
    dst_p = jnp.concatenate([dst, jnp.full((pad,), N, jnp.int32)])
    src3 = src_p.reshape(NW, NBLK, B)
    dst3 = dst_p.reshape(NW, NBLK, B)

    partials = _sc_segment_sum(x, src3, dst3)
    p0 = partials[:N]
    p1 = partials[N_PAD:N_PAD + N]
    return _tc_linear(p0, p1, W)


# speedup vs baseline: 2.7248x; 2.7248x over previous
import functools

import jax
import jax.numpy as jnp
from jax import lax
from jax.experimental import pallas as pl
from jax.experimental.pallas import tpu as pltpu
from jax.experimental.pallas import tpu_sc as plsc

N = 10000
E = 320000
D = 128

NC = 2
NS = 16
NW = NC * NS
B = 128
CH = 16
NBLK = 80
E_PAD = NW * B * NBLK
NCHUNK = NBLK // CH
N_PAD = 10240
RPT = N_PAD // NS


def _sc_segment_sum(x, src3, dst3):
    mesh = plsc.VectorSubcoreMesh(core_axis_name="c", subcore_axis_name="s")

    @functools.partial(
        pl.kernel,
        out_type=jax.ShapeDtypeStruct((NC * N_PAD, D), jnp.float32),
        mesh=mesh,
        scratch_types=dict(
            acc=pltpu.VMEM_SHARED((N_PAD, D), jnp.float32),
            sidx=pltpu.VMEM((2, CH, B), jnp.int32),
            didx=pltpu.VMEM((2, CH, B), jnp.int32),
            rows_v=pltpu.VMEM((2, B, D), jnp.float32),
            gsem=pltpu.SemaphoreType.DMA,
            isem=pltpu.SemaphoreType.DMA,
        ),
    )
    def seg_sum(x_hbm, src_hbm, dst_hbm, out_hbm, acc, sidx, didx,
                rows_v, gsem, isem):
        c = lax.axis_index("c")
        s = lax.axis_index("s")
        wid = c * NS + s

        z = jnp.zeros((16,), jnp.float32)

        def zrow(i, carry):
            for j in range(D // 16):
                rows_v[0, i, pl.ds(j * 16, 16)] = z
            return carry
        lax.fori_loop(0, B, zrow, 0)

        def zero_body(k, carry):
            pltpu.sync_copy(rows_v.at[0], acc.at[pl.ds(s * RPT + k * B, B)])
            return carry
        lax.fori_loop(0, RPT // B, zero_body, 0)

        pltpu.sync_copy(src_hbm.at[wid, pl.ds(0, CH)], sidx.at[0])
        pltpu.sync_copy(dst_hbm.at[wid, pl.ds(0, CH)], didx.at[0])

        plsc.subcore_barrier()

        def chunk_body(ci, carry):
            cb = lax.rem(ci, 2)
            nb = lax.rem(ci + 1, 2)

            @pl.when(ci + 1 < NCHUNK)
            def _():
                pltpu.async_copy(
                    src_hbm.at[wid, pl.ds((ci + 1) * CH, CH)],
                    sidx.at[nb], isem)
                pltpu.async_copy(
                    dst_hbm.at[wid, pl.ds((ci + 1) * CH, CH)],
                    didx.at[nb], isem)

            for j in range(CH):
                b0 = j % 2
                pltpu.async_copy(x_hbm.at[sidx.at[cb, j]], rows_v.at[b0],
                                 gsem).wait()
                pltpu.sync_copy(rows_v.at[b0], acc.at[didx.at[cb, j]],
                                add=True)

            @pl.when(ci + 1 < NCHUNK)
            def _():
                pltpu.make_async_copy(
                    src_hbm.at[wid, pl.ds(0, CH)], sidx.at[nb], isem).wait()
                pltpu.make_async_copy(
                    dst_hbm.at[wid, pl.ds(0, CH)], didx.at[nb], isem).wait()
            return carry
        lax.fori_loop(0, NCHUNK, chunk_body, 0)

        plsc.subcore_barrier()

        pltpu.sync_copy(acc.at[pl.ds(s * RPT, RPT)],
                        out_hbm.at[pl.ds(c * N_PAD + s * RPT, RPT)])

    return seg_sum(x, src3, dst3)


def _tc_linear(p0, p1, W):
    BN = 1000

    def body(p0_ref, p1_ref, w_ref, o_ref):
        agg = p0_ref[...] + p1_ref[...]
        o_ref[...] = lax.dot_general(
            agg, w_ref[...], (((1,), (1,)), ((), ())),
            preferred_element_type=jnp.float32)

    return pl.pallas_call(
        body,
        grid=(N // BN,),
        in_specs=[
            pl.BlockSpec((BN, D), lambda i: (i, 0)),
            pl.BlockSpec((BN, D), lambda i: (i, 0)),
            pl.BlockSpec((D, D), lambda i: (0, 0)),
        ],
        out_specs=pl.BlockSpec((BN, D), lambda i: (i, 0)),
        out_shape=jax.ShapeDtypeStruct((N, D), jnp.float32),
    )(p0, p1, W)


def kernel(x, edge_index, W):
    dst = edge_index[0]
    src = edge_index[1]
    pad = E_PAD - E
    ar = jax.lax.iota(jnp.int32, pad)
    src_p = jnp.concatenate([src, ar % N])
    dst_p = jnp.concatenate([dst, N + ar % (N_PAD - N)])
    src3 = src_p.reshape(NW, NBLK, B)
    dst3 = dst_p.reshape(NW, NBLK, B)

    partials = _sc_segment_sum(x, src3, dst3)
    p0 = partials[:N]
    p1 = partials[N_PAD:N_PAD + N]
    return _tc_linear(p0, p1, W)


# speedup vs baseline: 3.8699x; 1.4203x over previous
import functools

import jax
import jax.numpy as jnp
from jax import lax
from jax.experimental import pallas as pl
from jax.experimental.pallas import tpu as pltpu
from jax.experimental.pallas import tpu_sc as plsc

N = 10000
E = 320000
D = 128

NC = 2
NS = 16
NW = NC * NS
B = 128
CH = 16
NBLK = 80
E_PAD = NW * B * NBLK
NCHUNK = NBLK // CH
N_PAD = 10240
RPT = N_PAD // NS


def _sc_segment_sum(x, src3, dst3):
    mesh = plsc.VectorSubcoreMesh(core_axis_name="c", subcore_axis_name="s")

    @functools.partial(
        pl.kernel,
        out_type=jax.ShapeDtypeStruct((NC * N_PAD, D), jnp.float32),
        mesh=mesh,
        scratch_types=dict(
            acc=pltpu.VMEM_SHARED((N_PAD, D), jnp.float32),
            sidx=pltpu.VMEM((2, CH, B), jnp.int32),
            didx=pltpu.VMEM((2, CH, B), jnp.int32),
            rows_v=pltpu.VMEM((2, B, D), jnp.float32),
            gsem=pltpu.SemaphoreType.DMA,
            isem=pltpu.SemaphoreType.DMA,
        ),
    )
    def seg_sum(x_hbm, src_hbm, dst_hbm, out_hbm, acc, sidx, didx,
                rows_v, gsem, isem):
        c = lax.axis_index("c")
        s = lax.axis_index("s")
        wid = c * NS + s

        z = jnp.zeros((16,), jnp.float32)

        def zrow(i, carry):
            for j in range(D // 16):
                rows_v[0, i, pl.ds(j * 16, 16)] = z
            return carry
        lax.fori_loop(0, B, zrow, 0)

        def zero_body(k, carry):
            pltpu.sync_copy(rows_v.at[0], acc.at[pl.ds(s * RPT + k * B, B)])
            return carry
        lax.fori_loop(0, RPT // B, zero_body, 0)

        pltpu.sync_copy(src_hbm.at[wid, pl.ds(0, CH)], sidx.at[0])
        pltpu.sync_copy(dst_hbm.at[wid, pl.ds(0, CH)], didx.at[0])

        plsc.subcore_barrier()

        def chunk_body(ci, carry):
            cb = lax.rem(ci, 2)
            nb = lax.rem(ci + 1, 2)

            @pl.when(ci + 1 < NCHUNK)
            def _():
                pltpu.async_copy(
                    src_hbm.at[wid, pl.ds((ci + 1) * CH, CH)],
                    sidx.at[nb], isem)
                pltpu.async_copy(
                    dst_hbm.at[wid, pl.ds((ci + 1) * CH, CH)],
                    didx.at[nb], isem)

            pltpu.async_copy(x_hbm.at[sidx.at[cb, 0]], rows_v.at[0], gsem)
            for j in range(CH):
                b0 = j % 2
                if j + 1 < CH:
                    pltpu.async_copy(x_hbm.at[sidx.at[cb, j + 1]],
                                     rows_v.at[(j + 1) % 2], gsem)
                pltpu.make_async_copy(
                    x_hbm.at[sidx.at[cb, j]], rows_v.at[b0], gsem).wait()
                pltpu.sync_copy(rows_v.at[b0], acc.at[didx.at[cb, j]],
                                add=True)

            @pl.when(ci + 1 < NCHUNK)
            def _():
                pltpu.make_async_copy(
                    src_hbm.at[wid, pl.ds(0, CH)], sidx.at[nb], isem).wait()
                pltpu.make_async_copy(
                    dst_hbm.at[wid, pl.ds(0, CH)], didx.at[nb], isem).wait()
            return carry
        lax.fori_loop(0, NCHUNK, chunk_body, 0)

        plsc.subcore_barrier()

        pltpu.sync_copy(acc.at[pl.ds(s * RPT, RPT)],
                        out_hbm.at[pl.ds(c * N_PAD + s * RPT, RPT)])

    return seg_sum(x, src3, dst3)


def _tc_linear(p0, p1, W):
    BN = 1000

    def body(p0_ref, p1_ref, w_ref, o_ref):
        agg = p0_ref[...] + p1_ref[...]
        o_ref[...] = lax.dot_general(
            agg, w_ref[...], (((1,), (1,)), ((), ())),
            preferred_element_type=jnp.float32)

    return pl.pallas_call(
        body,
        grid=(N // BN,),
        in_specs=[
            pl.BlockSpec((BN, D), lambda i: (i, 0)),
            pl.BlockSpec((BN, D), lambda i: (i, 0)),
            pl.BlockSpec((D, D), lambda i: (0, 0)),
        ],
        out_specs=pl.BlockSpec((BN, D), lambda i: (i, 0)),
        out_shape=jax.ShapeDtypeStruct((N, D), jnp.float32),
    )(p0, p1, W)


def kernel(x, edge_index, W):
    dst = edge_index[0]
    src = edge_index[1]
    pad = E_PAD - E
    ar = jax.lax.iota(jnp.int32, pad)
    src_p = jnp.concatenate([src, ar % N])
    dst_p = jnp.concatenate([dst, N + ar % (N_PAD - N)])
    src3 = src_p.reshape(NW, NBLK, B)
    dst3 = dst_p.reshape(NW, NBLK, B)

    partials = _sc_segment_sum(x, src3, dst3)
    p0 = partials[:N]
    p1 = partials[N_PAD:N_PAD + N]
    return _tc_linear(p0, p1, W)
